# baseline (speedup 1.0000x reference)
import functools

import jax
import jax.numpy as jnp
import numpy as np
from jax import lax
from jax.experimental import pallas as pl
from jax.experimental.pallas import tpu as pltpu
from jax.experimental.pallas import tpu_sc as plsc

NUM_INTERVALS = 128
MAX_TIME = 1.0
D_FEAT = 128
N_ROWS = 131072

NC = 2
NS = 16
L = 16

CHUNK = 128
NGROUPS = CHUNK // L
NCHUNKS = N_ROWS // (NC * NS) // CHUNK


def _sc_body(t_hbm, z_hbm, ind_hbm, dt_hbm, dtind_hbm, tauind_hbm,
             taunext_hbm, vin, tin, dtbuf, ebuf, indbuf, dtindbuf,
             tauindbuf, taunextbuf, insem, outsem):
    c = lax.axis_index("c")
    s = lax.axis_index("s")
    wid = c * NS + s
    rows_per = N_ROWS // (NC * NS)
    base = wid * rows_per
    lane = lax.iota(jnp.int32, L)
    zero16f = jnp.zeros((L,), jnp.float32)
    zero16i = jnp.zeros((L,), jnp.int32)

    def in_copies(ci, p):
        row0 = base + ci * CHUNK
        return (
            pltpu.make_async_copy(
                z_hbm.at[pl.ds(row0, CHUNK), pl.ds(D_FEAT, NUM_INTERVALS)],
                vin.at[p], insem.at[p]),
            pltpu.make_async_copy(t_hbm.at[pl.ds(row0, CHUNK)], tin.at[p],
                                  insem.at[p]),
        )

    def out_copies(ci, p):
        row0 = base + ci * CHUNK
        dst = pl.ds(row0, CHUNK)
        return (
            pltpu.make_async_copy(dtbuf.at[p], dt_hbm.at[dst], outsem.at[p]),
            pltpu.make_async_copy(indbuf.at[p], ind_hbm.at[dst], outsem.at[p]),
            pltpu.make_async_copy(dtindbuf.at[p], dtind_hbm.at[dst],
                                  outsem.at[p]),
            pltpu.make_async_copy(tauindbuf.at[p], tauind_hbm.at[dst],
                                  outsem.at[p]),
            pltpu.make_async_copy(taunextbuf.at[p], taunext_hbm.at[dst],
                                  outsem.at[p]),
        )

    for cp in in_copies(0, 0):
        cp.start()
    for cp in in_copies(1, 1):
        cp.start()

    def chunk_body(ci, carry):
        p = jnp.bitwise_and(ci, 1)
        p16 = jnp.broadcast_to(p, (L,))
        for cp in in_copies(ci, p):
            cp.wait()

        @pl.when(ci >= 2)
        def _():
            for cp in out_copies(ci - 2, p):
                cp.wait()

        def make_a_step(ga, epar):
            rows16 = ga * L + lane

            def step(j, s_acc):
                col = plsc.load_gather(
                    vin, [p16, rows16, jnp.full((L,), j, jnp.int32)])
                e = jnp.exp(col)
                ebuf[epar, j] = e
                return s_acc + e

            return step

        def make_b_step(gb, epar, t16):
            rows16 = gb * L + lane

            def step(j, st):
                tau, ind_acc, seen, taunext, dtind, inv = st
                e = ebuf[epar, j]
                dt_col = e * inv
                plsc.store_scatter(
                    dtbuf, [p16, rows16, jnp.full((L,), j, jnp.int32)], dt_col)
                tau = tau + dt_col
                if j < NUM_INTERVALS - 1:
                    m = tau < t16
                    newly = jnp.logical_and(jnp.logical_not(m),
                                            jnp.logical_not(seen))
                else:
                    newly = jnp.logical_not(seen)
                jc = jnp.full((L,), j, jnp.int32)
                ind_acc = jnp.where(newly, jc, ind_acc)
                taunext = jnp.where(newly, tau, taunext)
                dtind = jnp.where(newly, dt_col, dtind)
                seen = jnp.logical_or(seen, newly)
                return (tau, ind_acc, seen, taunext, dtind, inv)

            return step

        def b_finish(gb, st):
            _, ind_acc, _, taunext, dtind, _ = st
            sl = pl.ds(gb * L, L)
            indbuf[p, sl] = ind_acc
            dtindbuf[p, sl] = dtind
            taunextbuf[p, sl] = taunext
            tauindbuf[p, sl] = taunext - dtind

        def b_init(s_acc):
            inv = 1.0 / s_acc
            false16 = zero16i > 0
            return (zero16f, zero16i, false16, zero16f, zero16f, inv)

        step_a0 = make_a_step(0, 0)
        s_acc = zero16f
        for j in range(NUM_INTERVALS):
            s_acc = step_a0(j, s_acc)

        def pipe_body(g, s_carry):
            epar = jnp.bitwise_and(g, 1)
            t16 = tin[p, pl.ds((g - 1) * L, L)]
            step_a = make_a_step(g, epar)
            step_b = make_b_step(g - 1, 1 - epar, t16)
            sa = zero16f
            stb = b_init(s_carry)
            for j in range(NUM_INTERVALS):
                sa = step_a(j, sa)
                stb = step_b(j, stb)
            b_finish(g - 1, stb)
            return sa

        s_acc = lax.fori_loop(1, NGROUPS, pipe_body, s_acc)

        glast = NGROUPS - 1
        t16 = tin[p, pl.ds(glast * L, L)]
        stb = b_init(s_acc)
        step_b = make_b_step(glast, glast & 1, t16)
        for j in range(NUM_INTERVALS):
            stb = step_b(j, stb)
        b_finish(glast, stb)

        for cp in out_copies(ci, p):
            cp.start()

        @pl.when(ci + 2 < NCHUNKS)
        def _():
            for cp in in_copies(ci + 2, p):
                cp.start()

        return carry

    lax.fori_loop(0, NCHUNKS, chunk_body, 0)

    for cp in out_copies(NCHUNKS - 2, 0):
        cp.wait()
    for cp in out_copies(NCHUNKS - 1, 1):
        cp.wait()


@jax.jit
def kernel(t, z):
    n = t.shape[0]
    mesh = plsc.VectorSubcoreMesh(core_axis_name="c", subcore_axis_name="s")
    out_type = (
        jax.ShapeDtypeStruct((n,), jnp.int32),
        jax.ShapeDtypeStruct((n, NUM_INTERVALS), jnp.float32),
        jax.ShapeDtypeStruct((n,), jnp.float32),
        jax.ShapeDtypeStruct((n,), jnp.float32),
        jax.ShapeDtypeStruct((n,), jnp.float32),
    )
    scratch = [
        pltpu.VMEM((2, CHUNK, NUM_INTERVALS), jnp.float32),
        pltpu.VMEM((2, CHUNK), jnp.float32),
        pltpu.VMEM((2, CHUNK, NUM_INTERVALS), jnp.float32),
        pltpu.VMEM((2, NUM_INTERVALS, L), jnp.float32),
        pltpu.VMEM((2, CHUNK), jnp.int32),
        pltpu.VMEM((2, CHUNK), jnp.float32),
        pltpu.VMEM((2, CHUNK), jnp.float32),
        pltpu.VMEM((2, CHUNK), jnp.float32),
        pltpu.SemaphoreType.DMA((2,)),
        pltpu.SemaphoreType.DMA((2,)),
    ]
    ind, dt, dt_ind, tau_ind, tau_next = pl.kernel(
        _sc_body,
        out_type=out_type,
        mesh=mesh,
        scratch_types=scratch,
        compiler_params=pltpu.CompilerParams(needs_layout_passes=False),
    )(t, z)
    z0 = z[:, :D_FEAT]
    return (ind, dt, dt_ind, tau_ind, tau_next, z0)


# speedup vs baseline: 3.8435x; 3.8435x over previous
import functools

import jax
import jax.numpy as jnp
import numpy as np
from jax import lax
from jax.experimental import pallas as pl
from jax.experimental.pallas import tpu as pltpu
from jax.experimental.pallas import tpu_sc as plsc

NUM_INTERVALS = 128
MAX_TIME = 1.0
D_FEAT = 128
N_ROWS = 131072

NC = 2
NS = 16
L = 16
NV = NUM_INTERVALS // L

CHUNK = 128
NCHUNKS = N_ROWS // (NC * NS) // CHUNK


def _sc_body(t_hbm, z_hbm, ind_hbm, dt_hbm, dtind_hbm, tauind_hbm,
             taunext_hbm, vin, tin, dtbuf, taubuf, indbuf, dtindbuf,
             tauindbuf, taunextbuf, insem, outsem):
    c = lax.axis_index("c")
    s = lax.axis_index("s")
    wid = c * NS + s
    rows_per = N_ROWS // (NC * NS)
    base = wid * rows_per
    lane = lax.iota(jnp.int32, L)

    def in_copies(ci, p):
        row0 = base + ci * CHUNK
        return (
            pltpu.make_async_copy(
                z_hbm.at[pl.ds(row0, CHUNK), pl.ds(D_FEAT, NUM_INTERVALS)],
                vin.at[p], insem.at[p]),
            pltpu.make_async_copy(t_hbm.at[pl.ds(row0, CHUNK)], tin.at[p],
                                  insem.at[p]),
        )

    def out_copies(ci, p):
        row0 = base + ci * CHUNK
        dst = pl.ds(row0, CHUNK)
        return (
            pltpu.make_async_copy(dtbuf.at[p], dt_hbm.at[dst], outsem.at[p]),
            pltpu.make_async_copy(indbuf.at[p], ind_hbm.at[dst], outsem.at[p]),
            pltpu.make_async_copy(dtindbuf.at[p], dtind_hbm.at[dst],
                                  outsem.at[p]),
            pltpu.make_async_copy(tauindbuf.at[p], tauind_hbm.at[dst],
                                  outsem.at[p]),
            pltpu.make_async_copy(taunextbuf.at[p], taunext_hbm.at[dst],
                                  outsem.at[p]),
        )

    for cp in in_copies(0, 0):
        cp.start()
    for cp in in_copies(1, 1):
        cp.start()

    def chunk_body(ci, carry):
        p = jnp.bitwise_and(ci, 1)
        for cp in in_copies(ci, p):
            cp.wait()

        @pl.when(ci >= 2)
        def _():
            for cp in out_copies(ci - 2, p):
                cp.wait()

        def group_body(g, gcarry):
            ind_acc = jnp.zeros((L,), jnp.int32)
            inv_acc = jnp.zeros((L,), jnp.float32)
            t16 = tin[p, pl.ds(g * L, L)]
            for j in range(L):
                r = g * L + j
                tj = t16[j]
                e = []
                cume = []
                for i in range(NV):
                    ei = jnp.exp(vin[p, r, pl.ds(i * L, L)])
                    e.append(ei)
                    cume.append(plsc.cumsum(ei))
                prefix = []
                tot = np.float32(0.0)
                for i in range(NV):
                    prefix.append(tot)
                    tot = tot + cume[i][L - 1]
                inv = 1.0 / jnp.broadcast_to(tot, (L,))
                tt16 = jnp.broadcast_to(tj, (L,)) * jnp.broadcast_to(tot, (L,))
                cnt = jnp.zeros((L,), jnp.int32)
                for i in range(NV):
                    dtbuf[p, r, pl.ds(i * L, L)] = e[i] * inv
                    ctau = cume[i] + prefix[i]
                    taubuf[j, pl.ds(i * L, L)] = ctau
                    m = ctau < tt16
                    if i == NV - 1:
                        m = m & (lane < L - 1)
                    cnt = cnt + m.astype(jnp.int32)
                indj = jnp.sum(cnt)
                ind_acc = jnp.where(lane == j, indj, ind_acc)
                inv_acc = jnp.where(lane == j, inv, inv_acc)
            rows16 = g * L + lane
            p16 = jnp.broadcast_to(p, (L,))
            dtind16 = plsc.load_gather(dtbuf, [p16, rows16, ind_acc])
            taunext16 = plsc.load_gather(taubuf, [lane, ind_acc]) * inv_acc
            indbuf[p, pl.ds(g * L, L)] = ind_acc
            dtindbuf[p, pl.ds(g * L, L)] = dtind16
            taunextbuf[p, pl.ds(g * L, L)] = taunext16
            tauindbuf[p, pl.ds(g * L, L)] = taunext16 - dtind16
            return gcarry

        lax.fori_loop(0, CHUNK // L, group_body, 0)

        for cp in out_copies(ci, p):
            cp.start()

        @pl.when(ci + 2 < NCHUNKS)
        def _():
            for cp in in_copies(ci + 2, p):
                cp.start()

        return carry

    lax.fori_loop(0, NCHUNKS, chunk_body, 0)

    for cp in out_copies(NCHUNKS - 2, 0):
        cp.wait()
    for cp in out_copies(NCHUNKS - 1, 1):
        cp.wait()


@jax.jit
def kernel(t, z):
    n = t.shape[0]
    mesh = plsc.VectorSubcoreMesh(core_axis_name="c", subcore_axis_name="s")
    out_type = (
        jax.ShapeDtypeStruct((n,), jnp.int32),
        jax.ShapeDtypeStruct((n, NUM_INTERVALS), jnp.float32),
        jax.ShapeDtypeStruct((n,), jnp.float32),
        jax.ShapeDtypeStruct((n,), jnp.float32),
        jax.ShapeDtypeStruct((n,), jnp.float32),
    )
    scratch = [
        pltpu.VMEM((2, CHUNK, NUM_INTERVALS), jnp.float32),
        pltpu.VMEM((2, CHUNK), jnp.float32),
        pltpu.VMEM((2, CHUNK, NUM_INTERVALS), jnp.float32),
        pltpu.VMEM((L, NUM_INTERVALS), jnp.float32),
        pltpu.VMEM((2, CHUNK), jnp.int32),
        pltpu.VMEM((2, CHUNK), jnp.float32),
        pltpu.VMEM((2, CHUNK), jnp.float32),
        pltpu.VMEM((2, CHUNK), jnp.float32),
        pltpu.SemaphoreType.DMA((2,)),
        pltpu.SemaphoreType.DMA((2,)),
    ]
    ind, dt, dt_ind, tau_ind, tau_next = pl.kernel(
        _sc_body,
        out_type=out_type,
        mesh=mesh,
        scratch_types=scratch,
        compiler_params=pltpu.CompilerParams(needs_layout_passes=False),
    )(t, z)
    z0 = z[:, :D_FEAT]
    return (ind, dt, dt_ind, tau_ind, tau_next, z0)
